# Initial kernel scaffold; baseline (speedup 1.0000x reference)
import jax
import jax.numpy as jnp
from jax.experimental import pallas as pl


def kernel(x, neutral_idx, W):
    raise NotImplementedError("write your pallas kernel here")


# speedup vs baseline: 1.0315x; 1.0315x over previous
import functools

import jax
import jax.numpy as jnp
from jax import lax
from jax.experimental import pallas as pl
from jax.experimental.pallas import tpu as pltpu
from jax.experimental.pallas import tpu_sc as plsc

_LN2 = 0.6931471805599453

_B = 16384
_L = 16
_NC, _NS = 2, 16
_NW = _NC * _NS
_BPW = _B // _NW
_GCHUNK = 128
_NG = _BPW // _GCHUNK


@functools.partial(
    pl.kernel,
    out_type=jax.ShapeDtypeStruct((_B,), jnp.float32),
    mesh=plsc.VectorSubcoreMesh(core_axis_name="c", subcore_axis_name="s"),
    scratch_types=[
        pltpu.VMEM((_BPW,), jnp.int32),
        pltpu.VMEM((_BPW,), jnp.float32),
        pltpu.VMEM((_L,), jnp.int32),
        pltpu.VMEM((_L,), jnp.float32),
        pltpu.SemaphoreType.DMA,
        pltpu.SemaphoreType.DMA,
    ],
)
def _gain_lookup(x_hbm, nidx_hbm, w_hbm, out_hbm,
                 idx_v, rows_v, nidx_v, nval_v, sem, nsem):
    wid = lax.axis_index("s") * _NC + lax.axis_index("c")
    base = wid * _BPW

    pltpu.sync_copy(x_hbm.at[pl.ds(base, _BPW)], idx_v)
    pltpu.sync_copy(nidx_hbm, nidx_v)

    ncopy = pltpu.async_copy(w_hbm.at[nidx_v], nval_v, nsem)
    copies = []
    for j in range(_NG):
        copies.append(pltpu.async_copy(
            w_hbm.at[idx_v.at[pl.ds(j * _GCHUNK, _GCHUNK)]],
            rows_v.at[pl.ds(j * _GCHUNK, _GCHUNK)], sem))
    ncopy.wait()
    n16 = nval_v[...]
    for c in copies:
        c.wait()

    for i in range(_BPW // _L):
        v = rows_v[pl.ds(i * _L, _L)]
        rows_v[pl.ds(i * _L, _L)] = jnp.exp((v - n16) * _LN2)

    pltpu.sync_copy(rows_v, out_hbm.at[pl.ds(base, _BPW)])


def kernel(x, neutral_idx, W):
    w_flat = W.reshape(W.shape[0])
    nidx = jnp.full((_L,), neutral_idx, jnp.int32)
    out = _gain_lookup(x.astype(jnp.int32), nidx, w_flat)
    return out.reshape(_B, 1)


# speedup vs baseline: 1.0493x; 1.0172x over previous
import functools

import jax
import jax.numpy as jnp
from jax import lax
from jax.experimental import pallas as pl
from jax.experimental.pallas import tpu as pltpu
from jax.experimental.pallas import tpu_sc as plsc

_LN2 = 0.6931471805599453

_B = 16384
_L = 16
_NC, _NS = 2, 16
_NW = _NC * _NS
_BPW = _B // _NW
_GCHUNK = 128
_NG = _BPW // _GCHUNK


@functools.partial(
    pl.kernel,
    out_type=jax.ShapeDtypeStruct((_B,), jnp.float32),
    mesh=plsc.VectorSubcoreMesh(core_axis_name="c", subcore_axis_name="s"),
    scratch_types=[
        pltpu.VMEM((_BPW,), jnp.int32),
        pltpu.VMEM((_BPW,), jnp.float32),
        pltpu.VMEM((_L,), jnp.int32),
        pltpu.VMEM((_L,), jnp.float32),
        pltpu.SemaphoreType.DMA,
        pltpu.SemaphoreType.DMA,
        [pltpu.SemaphoreType.DMA] * _NG,
        pltpu.SemaphoreType.DMA,
    ],
)
def _gain_lookup(x_hbm, nidx_hbm, w_hbm, out_hbm,
                 idx_v, rows_v, nidx_v, nval_v, sem_idx, sem_n, gsems, sem_o):
    wid = lax.axis_index("s") * _NC + lax.axis_index("c")
    base = wid * _BPW

    ca = pltpu.async_copy(x_hbm.at[pl.ds(base, _BPW)], idx_v, sem_idx)
    cb = pltpu.async_copy(nidx_hbm, nidx_v, sem_n)
    cb.wait()
    nc = pltpu.async_copy(w_hbm.at[nidx_v], nval_v, sem_n)
    ca.wait()
    gathers = []
    for j in range(_NG):
        gathers.append(pltpu.async_copy(
            w_hbm.at[idx_v.at[pl.ds(j * _GCHUNK, _GCHUNK)]],
            rows_v.at[pl.ds(j * _GCHUNK, _GCHUNK)], gsems[j]))
    nc.wait()
    n16 = nval_v[...]

    stores = []
    for j, g in enumerate(gathers):
        g.wait()
        for i in range(_GCHUNK // _L):
            o = j * _GCHUNK + i * _L
            v = rows_v[pl.ds(o, _L)]
            rows_v[pl.ds(o, _L)] = jnp.exp((v - n16) * _LN2)
        stores.append(pltpu.async_copy(
            rows_v.at[pl.ds(j * _GCHUNK, _GCHUNK)],
            out_hbm.at[pl.ds(base + j * _GCHUNK, _GCHUNK)], sem_o))
    for s in stores:
        s.wait()


def kernel(x, neutral_idx, W):
    w_flat = W.reshape(W.shape[0])
    nidx = jnp.full((_L,), neutral_idx, jnp.int32)
    out = _gain_lookup(x.astype(jnp.int32), nidx, w_flat)
    return out.reshape(_B, 1)


# speedup vs baseline: 1.0509x; 1.0016x over previous
import functools

import jax
import jax.numpy as jnp
from jax import lax
from jax.experimental import pallas as pl
from jax.experimental.pallas import tpu as pltpu
from jax.experimental.pallas import tpu_sc as plsc

_LN2 = 0.6931471805599453

_B = 16384
_L = 16
_NC, _NS = 2, 16
_NW = _NC * _NS
_BPW = _B // _NW
_GCHUNK = 128
_NG = _BPW // _GCHUNK


@functools.partial(
    pl.kernel,
    out_type=jax.ShapeDtypeStruct((_B,), jnp.float32),
    mesh=plsc.VectorSubcoreMesh(core_axis_name="c", subcore_axis_name="s"),
    scratch_types=[
        pltpu.VMEM((_BPW,), jnp.int32),
        pltpu.VMEM((_BPW,), jnp.float32),
        pltpu.VMEM((_BPW,), jnp.float32),
        pltpu.VMEM((_L,), jnp.int32),
        pltpu.VMEM((_L,), jnp.float32),
        [pltpu.SemaphoreType.DMA] * _NG,
        pltpu.SemaphoreType.DMA,
        [pltpu.SemaphoreType.DMA] * _NG,
        pltpu.SemaphoreType.DMA,
    ],
)
def _gain_lookup(x_hbm, nidx_hbm, w_hbm, out_hbm,
                 idx_v, rows_v, res_v, nidx_v, nval_v, isems, sem_n, gsems,
                 sem_o):
    wid = lax.axis_index("s") * _NC + lax.axis_index("c")
    base = wid * _BPW

    idx_copies = []
    for j in range(_NG):
        idx_copies.append(pltpu.async_copy(
            x_hbm.at[pl.ds(base + j * _GCHUNK, _GCHUNK)],
            idx_v.at[pl.ds(j * _GCHUNK, _GCHUNK)], isems[j]))
    cb = pltpu.async_copy(nidx_hbm, nidx_v, sem_n)
    cb.wait()
    nc = pltpu.async_copy(w_hbm.at[nidx_v], nval_v, sem_n)
    gathers = []
    for j in range(_NG):
        idx_copies[j].wait()
        gathers.append(pltpu.async_copy(
            w_hbm.at[idx_v.at[pl.ds(j * _GCHUNK, _GCHUNK)]],
            rows_v.at[pl.ds(j * _GCHUNK, _GCHUNK)], gsems[j]))
    nc.wait()
    n16 = nval_v[...]

    stores = []
    for j, g in enumerate(gathers):
        g.wait()
        for i in range(_GCHUNK // _L):
            o = j * _GCHUNK + i * _L
            res_v[pl.ds(o, _L)] = jnp.exp((rows_v[pl.ds(o, _L)] - n16) * _LN2)
        stores.append(pltpu.async_copy(
            res_v.at[pl.ds(j * _GCHUNK, _GCHUNK)],
            out_hbm.at[pl.ds(base + j * _GCHUNK, _GCHUNK)], sem_o))
    for s in stores:
        s.wait()


def kernel(x, neutral_idx, W):
    w_flat = W.reshape(W.shape[0])
    nidx = jnp.full((_L,), neutral_idx, jnp.int32)
    out = _gain_lookup(x.astype(jnp.int32), nidx, w_flat)
    return out.reshape(_B, 1)


# speedup vs baseline: 1.1590x; 1.1029x over previous
import functools

import jax
import jax.numpy as jnp
from jax import lax
from jax.experimental import pallas as pl
from jax.experimental.pallas import tpu as pltpu
from jax.experimental.pallas import tpu_sc as plsc

_LN2 = 0.6931471805599453

_B = 16384
_L = 16
_NC, _NS = 2, 16
_NW = _NC * _NS
_BPW = _B // _NW
_GCHUNK = 128
_NG = _BPW // _GCHUNK


@functools.partial(
    pl.kernel,
    out_type=jax.ShapeDtypeStruct((_B // _GCHUNK, _GCHUNK), jnp.float32),
    mesh=plsc.VectorSubcoreMesh(core_axis_name="c", subcore_axis_name="s"),
    scratch_types=[
        pltpu.VMEM((_NG, _GCHUNK), jnp.int32),
        pltpu.VMEM((_NG, _GCHUNK), jnp.float32),
        pltpu.VMEM((_NG, _GCHUNK), jnp.float32),
        pltpu.VMEM((1,), jnp.int32),
        pltpu.VMEM((_L,), jnp.float32),
        pltpu.SemaphoreType.DMA,
        pltpu.SemaphoreType.DMA,
        pltpu.SemaphoreType.DMA,
        pltpu.SemaphoreType.DMA,
    ],
)
def _gain_lookup(x_hbm, nidx_hbm, w_hbm, out_hbm,
                 idx_v, rows_v, res_v, nidx_v, nval_v,
                 sem_i, sem_n, sem_g, sem_o):
    wid = lax.axis_index("s") * _NC + lax.axis_index("c")
    base = wid * _NG

    ca = pltpu.async_copy(x_hbm.at[pl.ds(base, _NG)], idx_v, sem_i)
    cb = pltpu.async_copy(nidx_hbm, nidx_v, sem_n)
    cb.wait()
    nc = pltpu.async_copy(w_hbm.at[nidx_v], nval_v.at[pl.ds(0, 1)], sem_n)
    ca.wait()
    gathers = [
        pltpu.async_copy(w_hbm.at[idx_v.at[j]], rows_v.at[j], sem_g)
        for j in range(_NG)
    ]
    nc.wait()
    nv = nval_v[...]
    n16 = jnp.full((_L,), nv[0], jnp.float32)
    for g in gathers:
        g.wait()

    for j in range(_NG):
        for i in range(_GCHUNK // _L):
            s = pl.ds(i * _L, _L)
            res_v[j, s] = jnp.exp((rows_v[j, s] - n16) * _LN2)

    pltpu.async_copy(res_v, out_hbm.at[pl.ds(base, _NG)], sem_o).wait()


def kernel(x, neutral_idx, W):
    w_flat = W.reshape(W.shape[0])
    nidx = jnp.asarray(neutral_idx, jnp.int32).reshape(1)
    x2 = x.astype(jnp.int32).reshape(_B // _GCHUNK, _GCHUNK)
    out = _gain_lookup(x2, nidx, w_flat)
    return out.reshape(_B, 1)


# speedup vs baseline: 1.1611x; 1.0018x over previous
import functools

import jax
import jax.numpy as jnp
from jax import lax
from jax.experimental import pallas as pl
from jax.experimental.pallas import tpu as pltpu
from jax.experimental.pallas import tpu_sc as plsc

_LN2 = 0.6931471805599453

_B = 16384
_L = 16
_NC, _NS = 2, 16
_NW = _NC * _NS
_BPW = _B // _NW
_GCHUNK = 128
_NG = _BPW // _GCHUNK


@functools.partial(
    pl.kernel,
    out_type=jax.ShapeDtypeStruct((_B // _GCHUNK, _GCHUNK), jnp.float32),
    mesh=plsc.VectorSubcoreMesh(core_axis_name="c", subcore_axis_name="s"),
    scratch_types=[
        pltpu.VMEM((_NG, _GCHUNK), jnp.int32),
        pltpu.VMEM((_NG, _GCHUNK), jnp.float32),
        pltpu.VMEM((_NG, _GCHUNK), jnp.float32),
        pltpu.VMEM((1,), jnp.int32),
        pltpu.VMEM((_L,), jnp.float32),
        [pltpu.SemaphoreType.DMA] * 2,
        pltpu.SemaphoreType.DMA,
        [pltpu.SemaphoreType.DMA] * _NG,
        pltpu.SemaphoreType.DMA,
    ],
)
def _gain_lookup(x_hbm, nidx_hbm, w_hbm, out_hbm,
                 idx_v, rows_v, res_v, nidx_v, nval_v,
                 isems, sem_n, gsems, sem_o):
    wid = lax.axis_index("s") * _NC + lax.axis_index("c")
    base = wid * _NG

    ca = pltpu.async_copy(x_hbm.at[pl.ds(base, _NG)], idx_v, isems[0])
    cb = pltpu.async_copy(nidx_hbm, nidx_v, sem_n)
    cb.wait()
    nc = pltpu.async_copy(w_hbm.at[nidx_v], nval_v.at[pl.ds(0, 1)], sem_n)
    ca.wait()
    gathers = [
        pltpu.async_copy(w_hbm.at[idx_v.at[j]], rows_v.at[j], gsems[j])
        for j in range(_NG)
    ]
    nc.wait()
    nv = nval_v[...]
    n16 = jnp.full((_L,), nv[0], jnp.float32)

    for j in range(_NG):
        gathers[j].wait()
        for i in range(_GCHUNK // _L):
            s = pl.ds(i * _L, _L)
            res_v[j, s] = jnp.exp((rows_v[j, s] - n16) * _LN2)

    pltpu.async_copy(res_v, out_hbm.at[pl.ds(base, _NG)], sem_o).wait()


def kernel(x, neutral_idx, W):
    w_flat = W.reshape(W.shape[0])
    nidx = jnp.asarray(neutral_idx, jnp.int32).reshape(1)
    x2 = x.astype(jnp.int32).reshape(_B // _GCHUNK, _GCHUNK)
    out = _gain_lookup(x2, nidx, w_flat)
    return out.reshape(_B, 1)


# speedup vs baseline: 1.1674x; 1.0054x over previous
import functools

import jax
import jax.numpy as jnp
from jax import lax
from jax.experimental import pallas as pl
from jax.experimental.pallas import tpu as pltpu
from jax.experimental.pallas import tpu_sc as plsc

_LN2 = 0.6931471805599453

_B = 16384
_L = 16
_NC, _NS = 2, 16
_NW = _NC * _NS
_BPW = _B // _NW
_GCHUNK = 128
_NG = _BPW // _GCHUNK


@functools.partial(
    pl.kernel,
    out_type=jax.ShapeDtypeStruct((_B // _GCHUNK, _GCHUNK), jnp.float32),
    mesh=plsc.VectorSubcoreMesh(core_axis_name="c", subcore_axis_name="s"),
    scratch_types=[
        pltpu.VMEM((_NG, _GCHUNK), jnp.int32),
        pltpu.VMEM((_NG, _GCHUNK), jnp.float32),
        pltpu.VMEM((_NG, _GCHUNK), jnp.float32),
        pltpu.VMEM((1,), jnp.int32),
        pltpu.VMEM((_L,), jnp.float32),
        [pltpu.SemaphoreType.DMA] * 2,
        pltpu.SemaphoreType.DMA,
        [pltpu.SemaphoreType.DMA] * _NG,
        pltpu.SemaphoreType.DMA,
    ],
)
def _gain_lookup(x_hbm, nidx_hbm, w_hbm, out_hbm,
                 idx_v, rows_v, res_v, nidx_v, nval_v,
                 isems, sem_n, gsems, sem_o):
    wid = lax.axis_index("s") * _NC + lax.axis_index("c")
    base = wid * _NG

    ca = pltpu.async_copy(x_hbm.at[pl.ds(base, _NG)], idx_v, isems[0])
    cb = pltpu.async_copy(nidx_hbm, nidx_v, sem_n)
    cb.wait()
    nc = pltpu.async_copy(w_hbm.at[nidx_v], nval_v.at[pl.ds(0, 1)], sem_n)
    ca.wait()
    gathers = [
        pltpu.async_copy(w_hbm.at[idx_v.at[j]], rows_v.at[j], gsems[j])
        for j in range(_NG)
    ]
    nc.wait()
    nv = nval_v[...]
    n16 = jnp.full((_L,), nv[0], jnp.float32)
    for g in gathers:
        g.wait()

    for j in range(_NG):
        for i in range(_GCHUNK // _L):
            s = pl.ds(i * _L, _L)
            res_v[j, s] = jnp.exp((rows_v[j, s] - n16) * _LN2)

    pltpu.async_copy(res_v, out_hbm.at[pl.ds(base, _NG)], sem_o).wait()


def kernel(x, neutral_idx, W):
    w_flat = W.reshape(W.shape[0])
    nidx = jnp.asarray(neutral_idx, jnp.int32).reshape(1)
    x2 = x.astype(jnp.int32).reshape(_B // _GCHUNK, _GCHUNK)
    out = _gain_lookup(x2, nidx, w_flat)
    return out.reshape(_B, 1)
